# baseline (speedup 1.0000x reference)
import functools
import jax
import jax.numpy as jnp
import numpy as np
from jax import lax
from jax.experimental import pallas as pl
from jax.experimental.pallas import tpu as pltpu
from jax.experimental.pallas import tpu_sc as plsc

_LAMBDA_REG = 10.0
_LAMBDA_SIGMA = 1.0
_LAMBDA_KL = 0.05
_LAMBDA_KL_WARMUP = 0.005
_KNN_K = 5
_WARMUP_ITERS = 1000
_ANNEAL_ITERS = 3000
_PRIOR_DELTA_MIN = 0.5
_PRIOR_DELTA_MAX = 20.0
_LOG_SIGMA_MIN = -6.0
_LOG_SIGMA_MAX = 4.0

_BIG = np.float32(1e12)
_PEN_T2 = np.float32(1e-4)

_NC = 2
_NS = 16
_NW = _NC * _NS
_L = 16


_G = 32
_NCELL = _G * _G
_CELLW2 = np.float32(1.0 / (_G * _G))
_CELLW2_2 = np.float32(4.0 / (_G * _G))
_SENT = np.float32(3.0e38)
_CNTSZ = ((_NCELL + 1 + _L - 1) // _L) * _L
_SCAN_U = 8


def _insert5(ts, kf):
    t0, t1, t2, t3, t4 = ts
    n4 = jnp.minimum(jnp.maximum(kf, t3), t4)
    n3 = jnp.minimum(jnp.maximum(kf, t2), t3)
    n2 = jnp.minimum(jnp.maximum(kf, t1), t2)
    n1 = jnp.minimum(jnp.maximum(kf, t0), t1)
    n0 = jnp.minimum(kf, t0)
    return (n0, n1, n2, n3, n4)


def _knn_sc_call(qx, qy, kx, ky, n_pad, m_pad):
    qpw = n_pad // _NW
    ng = qpw // _L
    nkc = m_pad // _L
    bins_sz = m_pad + _L
    mesh = plsc.VectorSubcoreMesh(core_axis_name="c", subcore_axis_name="s")

    @functools.partial(
        pl.kernel,
        out_type=jax.ShapeDtypeStruct((_KNN_K * n_pad,), jnp.float32),
        mesh=mesh,
        compiler_params=pltpu.CompilerParams(needs_layout_passes=False),
        scratch_types=[
            pltpu.VMEM((m_pad,), jnp.float32),
            pltpu.VMEM((m_pad,), jnp.float32),
            pltpu.VMEM((qpw,), jnp.float32),
            pltpu.VMEM((qpw,), jnp.float32),
            pltpu.VMEM((_KNN_K * qpw,), jnp.float32),
            pltpu.VMEM((bins_sz,), jnp.float32),
            pltpu.VMEM((bins_sz,), jnp.float32),
            pltpu.VMEM((_CNTSZ,), jnp.int32),
            pltpu.VMEM((_CNTSZ,), jnp.int32),
            pltpu.VMEM((_CNTSZ,), jnp.int32),
        ],
    )
    def knn_kernel(qx_hbm, qy_hbm, kx_hbm, ky_hbm, out_hbm,
                   kx_v, ky_v, qx_v, qy_v, res_v, bx_v, by_v, cnt_v,
                   st_v, cur_v):
        wid = lax.axis_index("s") * _NC + lax.axis_index("c")
        pltpu.sync_copy(kx_hbm, kx_v)
        pltpu.sync_copy(ky_hbm, ky_v)
        base = wid * qpw
        pltpu.sync_copy(qx_hbm.at[pl.ds(base, qpw)], qx_v)
        pltpu.sync_copy(qy_hbm.at[pl.ds(base, qpw)], qy_v)

        zz = jnp.zeros((_L,), jnp.int32)
        for i in range(_CNTSZ // _L):
            cnt_v[pl.ds(i * _L, _L)] = zz

        gf = jnp.float32(_G)

        def key_cells(c):
            kxc = kx_v[pl.ds(c * _L, _L)]
            kyc = ky_v[pl.ds(c * _L, _L)]
            cxi = jnp.minimum(kxc * gf, 300.0).astype(jnp.int32)
            cyi = jnp.minimum(kyc * gf, 300.0).astype(jnp.int32)
            real = (cxi >= 0) & (cxi < _G) & (cyi >= 0) & (cyi < _G)
            cid = jnp.where(real, cyi * _G + cxi, _NCELL)
            return kxc, kyc, cid, real

        def cnt_body(c, carry):
            _, _, cid, real = key_cells(c)
            dup, last = plsc.scan_count(cid, mask=real)
            cnt = plsc.load_gather(cnt_v, [cid])
            plsc.store_scatter(cnt_v, [cid], cnt + dup, mask=last)
            return carry

        lax.fori_loop(0, nkc, cnt_body, 0)

        run = zz
        for i in range(_NCELL // _L):
            v = cnt_v[pl.ds(i * _L, _L)]
            cs = plsc.cumsum(v)
            st = cs - v + run
            st_v[pl.ds(i * _L, _L)] = st
            cur_v[pl.ds(i * _L, _L)] = st
            run = run + cs[_L - 1]
        st_v[pl.ds(_NCELL, _L)] = run

        def fill_body(c, carry):
            kxc, kyc, cid, real = key_cells(c)
            dup, last = plsc.scan_count(cid, mask=real)
            cur = plsc.load_gather(cur_v, [cid])
            pos = cur + (dup - 1)
            plsc.store_scatter(bx_v, [pos], kxc, mask=real)
            plsc.store_scatter(by_v, [pos], kyc, mask=real)
            plsc.store_scatter(cur_v, [cid], cur + dup, mask=last)
            return carry

        lax.fori_loop(0, nkc, fill_body, 0)

        def scan_range(ts, rbase, ln, qxg, qyg):
            maxln = jnp.max(ln)

            def sbody(it, ts):
                s0 = it * _SCAN_U
                for u in range(_SCAN_U):
                    s = s0 + u
                    m = ln > s
                    idx = jnp.minimum(rbase + s, bins_sz - 1)
                    bxv = plsc.load_gather(bx_v, [idx], mask=m)
                    byv = plsc.load_gather(by_v, [idx], mask=m)
                    dx = qxg - bxv
                    dy = qyg - byv
                    d2 = dx * dx + dy * dy
                    kf = jnp.where(d2 < _PEN_T2, _BIG, d2)
                    kf = jnp.where(m, kf, _SENT)
                    ts = _insert5(ts, kf)
                return ts

            return lax.fori_loop(0, (maxln + _SCAN_U - 1) // _SCAN_U,
                                 sbody, ts)

        def brute_group(qxg, qyg):
            def chunk_body(c, ts):
                kxc = kx_v[pl.ds(c * _L, _L)]
                kyc = ky_v[pl.ds(c * _L, _L)]
                for i in range(_L):
                    dx = qxg - kxc[i]
                    dy = qyg - kyc[i]
                    d2 = dx * dx + dy * dy
                    kf = jnp.where(d2 < _PEN_T2, _BIG, d2)
                    ts = _insert5(ts, kf)
                return ts

            init = tuple(jnp.full((_L,), _SENT, jnp.float32)
                         for _ in range(_KNN_K))
            return lax.fori_loop(0, nkc, chunk_body, init)

        def group_body(g, carry):
            qxg = qx_v[pl.ds(g * _L, _L)]
            qyg = qy_v[pl.ds(g * _L, _L)]
            cxi = jnp.clip((qxg * gf).astype(jnp.int32), 0, _G - 1)
            cyi = jnp.clip((qyg * gf).astype(jnp.int32), 0, _G - 1)
            ts = tuple(jnp.full((_L,), _SENT, jnp.float32)
                       for _ in range(_KNN_K))
            def scan_row(ts, rr, c0, c1, cvalid=None):
                rvalid = (rr >= 0) & (rr < _G)
                if cvalid is not None:
                    rvalid = rvalid & cvalid
                rrc = jnp.where(rvalid, rr, 0) * _G
                lo = plsc.load_gather(st_v, [rrc + c0])
                hi = plsc.load_gather(st_v, [rrc + c1 + 1])
                ln = jnp.where(rvalid, hi - lo, 0)
                return scan_range(ts, lo, ln, qxg, qyg)

            c0 = jnp.maximum(cxi - 1, 0)
            c1 = jnp.minimum(cxi + 1, _G - 1)
            for dr in (-1, 0, 1):
                ts = scan_row(ts, cyi + dr, c0, c1)

            fail1 = jnp.any(ts[_KNN_K - 1] > _CELLW2)

            def rescue(t):
                c0r = jnp.maximum(cxi - 2, 0)
                c1r = jnp.minimum(cxi + 2, _G - 1)
                for dr in (-2, 2):
                    t = scan_row(t, cyi + dr, c0r, c1r)
                for dr in (-1, 0, 1):
                    for cc in (cxi - 2, cxi + 2):
                        cv = (cc >= 0) & (cc < _G)
                        ccc = jnp.clip(cc, 0, _G - 1)
                        t = scan_row(t, cyi + dr, ccc, ccc, cvalid=cv)
                return t

            ts = lax.cond(fail1, rescue, lambda t: t, ts)
            fail2 = jnp.any(ts[_KNN_K - 1] > _CELLW2_2)
            ts = lax.cond(fail2, lambda t: brute_group(qxg, qyg),
                          lambda t: t, ts)
            for i in range(_KNN_K):
                res_v[pl.ds(i * qpw + g * _L, _L)] = ts[i]
            return carry

        lax.fori_loop(0, ng, group_body, 0)
        for i in range(_KNN_K):
            pltpu.sync_copy(res_v.at[pl.ds(i * qpw, qpw)],
                            out_hbm.at[pl.ds(i * n_pad + base, qpw)])

    return knn_kernel(qx, qy, kx, ky)


def _make_loss_tc_kernel(n, rows):
    def loss_tc_kernel(x_ref, t_ref, reg_ref, sig_ref, kl_ref):
        lanes = 128
        ridx = lax.broadcasted_iota(jnp.int32, (rows, lanes), 0)
        cidx = lax.broadcasted_iota(jnp.int32, (rows, lanes), 1)
        valid = (ridx * lanes + cidx) < n
        nf = jnp.float32(n)

        def row(i):
            return x_ref[i * rows:(i + 1) * rows, :]

        pdx, pdy = row(0), row(1)
        lsx, lsy = row(2), row(3)
        gx, gy = row(4), row(5)
        px, py = row(6), row(7)
        st = row(8)
        beta = jnp.float32(0.5)

        reg_sum = jnp.float32(0.0)
        sig_sum = jnp.float32(0.0)
        for pd, ls, g, p in ((pdx, lsx, gx, px), (pdy, lsy, gy, py)):
            lsc = jnp.clip(ls, _LOG_SIGMA_MIN, _LOG_SIGMA_MAX)
            sq = jnp.exp(lsc)
            gd = (g - p) / st
            diff = pd - gd
            ad = jnp.abs(diff)
            sl1 = jnp.where(ad < beta, 0.5 * diff * diff / beta,
                            ad - 0.5 * beta)
            reg_sum = reg_sum + jnp.sum(jnp.where(valid, sl1, 0.0))
            sigt = lsc + (diff * diff) / (2.0 * sq * sq)
            sig_sum = sig_sum + jnp.sum(jnp.where(valid, sigt, 0.0))

        dsum = jnp.zeros((rows, lanes), jnp.float32)
        for i in range(_KNN_K):
            s = t_ref[i * rows:(i + 1) * rows, :]
            d = jnp.sqrt(jnp.maximum(s, 1e-12))
            dsum = dsum + jnp.where(s >= 1e11, jnp.float32(1e8), d)
        d_i = dsum / jnp.float32(_KNN_K)
        d_norm = jnp.clip(d_i / st, _PRIOR_DELTA_MIN, _PRIOR_DELTA_MAX)
        sigma_c = jnp.maximum(d_norm, 0.5)
        sigma_p = jnp.maximum(sigma_c, 0.0001)

        kl_sum = jnp.float32(0.0)
        for pd, ls in ((pdx, lsx), (pdy, lsy)):
            lsc = jnp.clip(ls, _LOG_SIGMA_MIN, _LOG_SIGMA_MAX)
            sq = jnp.exp(lsc)
            kl = (jnp.log(sigma_p / sq)
                  + (sq * sq + pd * pd) / (2.0 * sigma_p * sigma_p) - 0.5)
            kl_sum = kl_sum + jnp.sum(jnp.where(valid, kl, 0.0))

        reg_ref[0, 0] = reg_sum / nf
        sig_ref[0, 0] = sig_sum / nf
        kl_ref[0, 0] = kl_sum / nf

    return loss_tc_kernel


def _pad_col(v, n_pad):
    n = v.shape[0]
    if n_pad != n:
        v = jnp.pad(v, (0, n_pad - n))
    return v


def kernel(pred_delta, pred_log_sigma, pos_points, pos_strides, gt_centers,
           gt_centers_list, cur_iter):
    n = pred_delta.shape[0]
    all_gt = gt_centers_list.reshape(-1, 2)
    m = all_gt.shape[0]

    n_pad = ((n + _NW * _L - 1) // (_NW * _L)) * (_NW * _L)
    n_pad = ((n_pad + 1023) // 1024) * 1024
    qx = jnp.pad(gt_centers[:, 0], (0, n_pad - n), constant_values=0.5)
    qy = jnp.pad(gt_centers[:, 1], (0, n_pad - n), constant_values=0.5)
    m_pad = ((m + _L - 1) // _L) * _L
    kx = jnp.pad(all_gt[:, 0], (0, m_pad - m), constant_values=1e9)
    ky = jnp.pad(all_gt[:, 1], (0, m_pad - m), constant_values=1e9)
    top5 = _knn_sc_call(qx, qy, kx, ky, n_pad, m_pad)

    rows = n_pad // 128
    xstk = jnp.stack([
        _pad_col(pred_delta[:, 0], n_pad),
        _pad_col(pred_delta[:, 1], n_pad),
        _pad_col(pred_log_sigma[:, 0], n_pad),
        _pad_col(pred_log_sigma[:, 1], n_pad),
        _pad_col(gt_centers[:, 0], n_pad),
        _pad_col(gt_centers[:, 1], n_pad),
        _pad_col(pos_points[:, 0], n_pad),
        _pad_col(pos_points[:, 1], n_pad),
        _pad_col(pos_strides.astype(jnp.float32), n_pad),
    ]).reshape(9 * rows, 128)
    t5 = top5.reshape(_KNN_K * rows, 128)
    scalar_spec = pl.BlockSpec(memory_space=pltpu.SMEM)
    reg, sig, kl = pl.pallas_call(
        _make_loss_tc_kernel(n, rows),
        out_shape=[jax.ShapeDtypeStruct((1, 1), jnp.float32)] * 3,
        in_specs=[pl.BlockSpec(memory_space=pltpu.VMEM)] * 2,
        out_specs=[scalar_spec] * 3,
    )(xstk, t5)

    cur = jnp.asarray(cur_iter, dtype=jnp.float32)
    ratio = jnp.minimum(1.0, (cur - _WARMUP_ITERS) / max(_ANNEAL_ITERS, 1))
    val = _LAMBDA_KL_WARMUP + ratio * (_LAMBDA_KL - _LAMBDA_KL_WARMUP)
    eff_lambda = jnp.where(cur < _WARMUP_ITERS, _LAMBDA_KL_WARMUP,
                           val).astype(jnp.float32)

    return (_LAMBDA_REG * reg[0, 0], _LAMBDA_SIGMA * sig[0, 0],
            eff_lambda * kl[0, 0])


# speedup vs baseline: 1.1018x; 1.1018x over previous
import functools
import jax
import jax.numpy as jnp
import numpy as np
from jax import lax
from jax.experimental import pallas as pl
from jax.experimental.pallas import tpu as pltpu
from jax.experimental.pallas import tpu_sc as plsc

_LAMBDA_REG = 10.0
_LAMBDA_SIGMA = 1.0
_LAMBDA_KL = 0.05
_LAMBDA_KL_WARMUP = 0.005
_KNN_K = 5
_WARMUP_ITERS = 1000
_ANNEAL_ITERS = 3000
_PRIOR_DELTA_MIN = 0.5
_PRIOR_DELTA_MAX = 20.0
_LOG_SIGMA_MIN = -6.0
_LOG_SIGMA_MAX = 4.0

_BIG = np.float32(1e12)
_PEN_T2 = np.float32(1e-4)

_NC = 2
_NS = 16
_NW = _NC * _NS
_L = 16


_G = 32
_NCELL = _G * _G
_CELLW2 = np.float32(1.0 / (_G * _G))
_CELLW2_2 = np.float32(4.0 / (_G * _G))
_SENT = np.float32(3.0e38)
_CNTSZ = ((_NCELL + 1 + _L - 1) // _L) * _L
_SCAN_U = 4


def _insert5(ts, kf):
    t0, t1, t2, t3, t4 = ts
    n4 = jnp.minimum(jnp.maximum(kf, t3), t4)
    n3 = jnp.minimum(jnp.maximum(kf, t2), t3)
    n2 = jnp.minimum(jnp.maximum(kf, t1), t2)
    n1 = jnp.minimum(jnp.maximum(kf, t0), t1)
    n0 = jnp.minimum(kf, t0)
    return (n0, n1, n2, n3, n4)


def _knn_sc_call(qx, qy, kx, ky, n_pad, m_pad):
    qpw = n_pad // _NW
    ng = qpw // _L
    nkc = m_pad // _L
    bins_sz = m_pad + _L
    mesh = plsc.VectorSubcoreMesh(core_axis_name="c", subcore_axis_name="s")

    @functools.partial(
        pl.kernel,
        out_type=jax.ShapeDtypeStruct((_KNN_K * n_pad,), jnp.float32),
        mesh=mesh,
        compiler_params=pltpu.CompilerParams(needs_layout_passes=False),
        scratch_types=[
            pltpu.VMEM((m_pad,), jnp.float32),
            pltpu.VMEM((m_pad,), jnp.float32),
            pltpu.VMEM((qpw,), jnp.float32),
            pltpu.VMEM((qpw,), jnp.float32),
            pltpu.VMEM((_KNN_K * qpw,), jnp.float32),
            pltpu.VMEM((bins_sz,), jnp.float32),
            pltpu.VMEM((bins_sz,), jnp.float32),
            pltpu.VMEM((_CNTSZ,), jnp.int32),
            pltpu.VMEM((_CNTSZ,), jnp.int32),
            pltpu.VMEM((_CNTSZ,), jnp.int32),
        ],
    )
    def knn_kernel(qx_hbm, qy_hbm, kx_hbm, ky_hbm, out_hbm,
                   kx_v, ky_v, qx_v, qy_v, res_v, bx_v, by_v, cnt_v,
                   st_v, cur_v):
        wid = lax.axis_index("s") * _NC + lax.axis_index("c")
        pltpu.sync_copy(kx_hbm, kx_v)
        pltpu.sync_copy(ky_hbm, ky_v)
        base = wid * qpw
        pltpu.sync_copy(qx_hbm.at[pl.ds(base, qpw)], qx_v)
        pltpu.sync_copy(qy_hbm.at[pl.ds(base, qpw)], qy_v)

        zz = jnp.zeros((_L,), jnp.int32)
        for i in range(_CNTSZ // _L):
            cnt_v[pl.ds(i * _L, _L)] = zz

        gf = jnp.float32(_G)

        def key_cells(c):
            kxc = kx_v[pl.ds(c * _L, _L)]
            kyc = ky_v[pl.ds(c * _L, _L)]
            cxi = jnp.minimum(kxc * gf, 300.0).astype(jnp.int32)
            cyi = jnp.minimum(kyc * gf, 300.0).astype(jnp.int32)
            real = (cxi >= 0) & (cxi < _G) & (cyi >= 0) & (cyi < _G)
            cid = jnp.where(real, cyi * _G + cxi, _NCELL)
            return kxc, kyc, cid, real

        def cnt_body(c, carry):
            _, _, cid, real = key_cells(c)
            dup, last = plsc.scan_count(cid, mask=real)
            cnt = plsc.load_gather(cnt_v, [cid])
            plsc.store_scatter(cnt_v, [cid], cnt + dup, mask=last)
            return carry

        lax.fori_loop(0, nkc, cnt_body, 0)

        run = zz
        for i in range(_NCELL // _L):
            v = cnt_v[pl.ds(i * _L, _L)]
            cs = plsc.cumsum(v)
            st = cs - v + run
            st_v[pl.ds(i * _L, _L)] = st
            cur_v[pl.ds(i * _L, _L)] = st
            run = run + cs[_L - 1]
        st_v[pl.ds(_NCELL, _L)] = run

        def fill_body(c, carry):
            kxc, kyc, cid, real = key_cells(c)
            dup, last = plsc.scan_count(cid, mask=real)
            cur = plsc.load_gather(cur_v, [cid])
            pos = cur + (dup - 1)
            plsc.store_scatter(bx_v, [pos], kxc, mask=real)
            plsc.store_scatter(by_v, [pos], kyc, mask=real)
            plsc.store_scatter(cur_v, [cid], cur + dup, mask=last)
            return carry

        lax.fori_loop(0, nkc, fill_body, 0)

        def scan_range(ts, rbase, ln, qxg, qyg):
            maxln = jnp.max(ln)

            def sbody(it, ts):
                s0 = it * _SCAN_U
                for u in range(_SCAN_U):
                    s = s0 + u
                    m = ln > s
                    idx = jnp.minimum(rbase + s, bins_sz - 1)
                    bxv = plsc.load_gather(bx_v, [idx], mask=m)
                    byv = plsc.load_gather(by_v, [idx], mask=m)
                    dx = qxg - bxv
                    dy = qyg - byv
                    d2 = dx * dx + dy * dy
                    kf = jnp.where(d2 < _PEN_T2, _BIG, d2)
                    kf = jnp.where(m, kf, _SENT)
                    ts = _insert5(ts, kf)
                return ts

            return lax.fori_loop(0, (maxln + _SCAN_U - 1) // _SCAN_U,
                                 sbody, ts)

        def brute_group(qxg, qyg):
            def chunk_body(c, ts):
                kxc = kx_v[pl.ds(c * _L, _L)]
                kyc = ky_v[pl.ds(c * _L, _L)]
                for i in range(_L):
                    dx = qxg - kxc[i]
                    dy = qyg - kyc[i]
                    d2 = dx * dx + dy * dy
                    kf = jnp.where(d2 < _PEN_T2, _BIG, d2)
                    ts = _insert5(ts, kf)
                return ts

            init = tuple(jnp.full((_L,), _SENT, jnp.float32)
                         for _ in range(_KNN_K))
            return lax.fori_loop(0, nkc, chunk_body, init)

        def group_body(g, carry):
            qxg = qx_v[pl.ds(g * _L, _L)]
            qyg = qy_v[pl.ds(g * _L, _L)]
            cxi = jnp.clip((qxg * gf).astype(jnp.int32), 0, _G - 1)
            cyi = jnp.clip((qyg * gf).astype(jnp.int32), 0, _G - 1)
            ts = tuple(jnp.full((_L,), _SENT, jnp.float32)
                       for _ in range(_KNN_K))
            def scan_row(ts, rr, c0, c1, cvalid=None):
                rvalid = (rr >= 0) & (rr < _G)
                if cvalid is not None:
                    rvalid = rvalid & cvalid
                rrc = jnp.where(rvalid, rr, 0) * _G
                lo = plsc.load_gather(st_v, [rrc + c0])
                hi = plsc.load_gather(st_v, [rrc + c1 + 1])
                ln = jnp.where(rvalid, hi - lo, 0)
                return scan_range(ts, lo, ln, qxg, qyg)

            c0 = jnp.maximum(cxi - 1, 0)
            c1 = jnp.minimum(cxi + 1, _G - 1)
            for dr in (-1, 0, 1):
                ts = scan_row(ts, cyi + dr, c0, c1)

            fail1 = jnp.any(ts[_KNN_K - 1] > _CELLW2)

            def rescue(t):
                c0r = jnp.maximum(cxi - 2, 0)
                c1r = jnp.minimum(cxi + 2, _G - 1)
                for dr in (-2, 2):
                    t = scan_row(t, cyi + dr, c0r, c1r)
                for dr in (-1, 0, 1):
                    for cc in (cxi - 2, cxi + 2):
                        cv = (cc >= 0) & (cc < _G)
                        ccc = jnp.clip(cc, 0, _G - 1)
                        t = scan_row(t, cyi + dr, ccc, ccc, cvalid=cv)
                return t

            ts = lax.cond(fail1, rescue, lambda t: t, ts)
            fail2 = jnp.any(ts[_KNN_K - 1] > _CELLW2_2)
            ts = lax.cond(fail2, lambda t: brute_group(qxg, qyg),
                          lambda t: t, ts)
            for i in range(_KNN_K):
                res_v[pl.ds(i * qpw + g * _L, _L)] = ts[i]
            return carry

        lax.fori_loop(0, ng, group_body, 0)
        for i in range(_KNN_K):
            pltpu.sync_copy(res_v.at[pl.ds(i * qpw, qpw)],
                            out_hbm.at[pl.ds(i * n_pad + base, qpw)])

    return knn_kernel(qx, qy, kx, ky)


def _make_loss_tc_kernels(n, rows):
    lanes = 128

    def valid_mask():
        ridx = lax.broadcasted_iota(jnp.int32, (rows, lanes), 0)
        cidx = lax.broadcasted_iota(jnp.int32, (rows, lanes), 1)
        return (ridx * lanes + cidx) < n

    def row(x_ref, i):
        return x_ref[i * rows:(i + 1) * rows, :]

    def loss_pre_kernel(x_ref, reg_ref, sig_ref):
        valid = valid_mask()
        nf = jnp.float32(n)
        st = row(x_ref, 8)
        beta = jnp.float32(0.5)
        reg_sum = jnp.float32(0.0)
        sig_sum = jnp.float32(0.0)
        for pd, ls, g, p in ((row(x_ref, 0), row(x_ref, 2), row(x_ref, 4),
                              row(x_ref, 6)),
                             (row(x_ref, 1), row(x_ref, 3), row(x_ref, 5),
                              row(x_ref, 7))):
            lsc = jnp.clip(ls, _LOG_SIGMA_MIN, _LOG_SIGMA_MAX)
            sq = jnp.exp(lsc)
            gd = (g - p) / st
            diff = pd - gd
            ad = jnp.abs(diff)
            sl1 = jnp.where(ad < beta, 0.5 * diff * diff / beta,
                            ad - 0.5 * beta)
            reg_sum = reg_sum + jnp.sum(jnp.where(valid, sl1, 0.0))
            sigt = lsc + (diff * diff) / (2.0 * sq * sq)
            sig_sum = sig_sum + jnp.sum(jnp.where(valid, sigt, 0.0))
        reg_ref[0, 0] = reg_sum / nf
        sig_ref[0, 0] = sig_sum / nf

    def loss_kl_kernel(x_ref, t_ref, kl_ref):
        valid = valid_mask()
        nf = jnp.float32(n)
        st = row(x_ref, 8)
        dsum = jnp.zeros((rows, lanes), jnp.float32)
        for i in range(_KNN_K):
            s = t_ref[i * rows:(i + 1) * rows, :]
            d = jnp.sqrt(jnp.maximum(s, 1e-12))
            dsum = dsum + jnp.where(s >= 1e11, jnp.float32(1e8), d)
        d_i = dsum / jnp.float32(_KNN_K)
        d_norm = jnp.clip(d_i / st, _PRIOR_DELTA_MIN, _PRIOR_DELTA_MAX)
        sigma_c = jnp.maximum(d_norm, 0.5)
        sigma_p = jnp.maximum(sigma_c, 0.0001)
        kl_sum = jnp.float32(0.0)
        for pd, ls in ((row(x_ref, 0), row(x_ref, 2)),
                       (row(x_ref, 1), row(x_ref, 3))):
            lsc = jnp.clip(ls, _LOG_SIGMA_MIN, _LOG_SIGMA_MAX)
            sq = jnp.exp(lsc)
            kl = (jnp.log(sigma_p / sq)
                  + (sq * sq + pd * pd) / (2.0 * sigma_p * sigma_p) - 0.5)
            kl_sum = kl_sum + jnp.sum(jnp.where(valid, kl, 0.0))
        kl_ref[0, 0] = kl_sum / nf

    return loss_pre_kernel, loss_kl_kernel


def _pad_col(v, n_pad):
    n = v.shape[0]
    if n_pad != n:
        v = jnp.pad(v, (0, n_pad - n))
    return v


def kernel(pred_delta, pred_log_sigma, pos_points, pos_strides, gt_centers,
           gt_centers_list, cur_iter):
    n = pred_delta.shape[0]
    all_gt = gt_centers_list.reshape(-1, 2)
    m = all_gt.shape[0]

    n_pad = ((n + _NW * _L - 1) // (_NW * _L)) * (_NW * _L)
    n_pad = ((n_pad + 1023) // 1024) * 1024
    qx = jnp.pad(gt_centers[:, 0], (0, n_pad - n), constant_values=0.5)
    qy = jnp.pad(gt_centers[:, 1], (0, n_pad - n), constant_values=0.5)
    m_pad = ((m + _L - 1) // _L) * _L
    kx = jnp.pad(all_gt[:, 0], (0, m_pad - m), constant_values=1e9)
    ky = jnp.pad(all_gt[:, 1], (0, m_pad - m), constant_values=1e9)
    top5 = _knn_sc_call(qx, qy, kx, ky, n_pad, m_pad)

    rows = n_pad // 128
    xstk = jnp.stack([
        _pad_col(pred_delta[:, 0], n_pad),
        _pad_col(pred_delta[:, 1], n_pad),
        _pad_col(pred_log_sigma[:, 0], n_pad),
        _pad_col(pred_log_sigma[:, 1], n_pad),
        _pad_col(gt_centers[:, 0], n_pad),
        _pad_col(gt_centers[:, 1], n_pad),
        _pad_col(pos_points[:, 0], n_pad),
        _pad_col(pos_points[:, 1], n_pad),
        _pad_col(pos_strides.astype(jnp.float32), n_pad),
    ]).reshape(9 * rows, 128)
    t5 = top5.reshape(_KNN_K * rows, 128)
    scalar_spec = pl.BlockSpec(memory_space=pltpu.SMEM)
    pre_k, kl_k = _make_loss_tc_kernels(n, rows)
    reg, sig = pl.pallas_call(
        pre_k,
        out_shape=[jax.ShapeDtypeStruct((1, 1), jnp.float32)] * 2,
        in_specs=[pl.BlockSpec(memory_space=pltpu.VMEM)],
        out_specs=[scalar_spec] * 2,
    )(xstk)
    (kl,) = pl.pallas_call(
        kl_k,
        out_shape=[jax.ShapeDtypeStruct((1, 1), jnp.float32)],
        in_specs=[pl.BlockSpec(memory_space=pltpu.VMEM)] * 2,
        out_specs=[scalar_spec],
    )(xstk, t5)

    cur = jnp.asarray(cur_iter, dtype=jnp.float32)
    ratio = jnp.minimum(1.0, (cur - _WARMUP_ITERS) / max(_ANNEAL_ITERS, 1))
    val = _LAMBDA_KL_WARMUP + ratio * (_LAMBDA_KL - _LAMBDA_KL_WARMUP)
    eff_lambda = jnp.where(cur < _WARMUP_ITERS, _LAMBDA_KL_WARMUP,
                           val).astype(jnp.float32)

    return (_LAMBDA_REG * reg[0, 0], _LAMBDA_SIGMA * sig[0, 0],
            eff_lambda * kl[0, 0])


# speedup vs baseline: 1.1504x; 1.0441x over previous
import functools
import jax
import jax.numpy as jnp
import numpy as np
from jax import lax
from jax.experimental import pallas as pl
from jax.experimental.pallas import tpu as pltpu
from jax.experimental.pallas import tpu_sc as plsc

_LAMBDA_REG = 10.0
_LAMBDA_SIGMA = 1.0
_LAMBDA_KL = 0.05
_LAMBDA_KL_WARMUP = 0.005
_KNN_K = 5
_WARMUP_ITERS = 1000
_ANNEAL_ITERS = 3000
_PRIOR_DELTA_MIN = 0.5
_PRIOR_DELTA_MAX = 20.0
_LOG_SIGMA_MIN = -6.0
_LOG_SIGMA_MAX = 4.0

_BIG = np.float32(1e12)
_PEN_T2 = np.float32(1e-4)

_NC = 2
_NS = 16
_NW = _NC * _NS
_L = 16


_G = 32
_NCELL = _G * _G
_CELLW2 = np.float32(1.0 / (_G * _G))
_CELLW2_2 = np.float32(4.0 / (_G * _G))
_SENT = np.float32(3.0e38)
_CNTSZ = ((_NCELL + 1 + _L - 1) // _L) * _L
_SCAN_U = 4


def _insert5(ts, kf):
    t0, t1, t2, t3, t4 = ts
    n4 = jnp.minimum(jnp.maximum(kf, t3), t4)
    n3 = jnp.minimum(jnp.maximum(kf, t2), t3)
    n2 = jnp.minimum(jnp.maximum(kf, t1), t2)
    n1 = jnp.minimum(jnp.maximum(kf, t0), t1)
    n0 = jnp.minimum(kf, t0)
    return (n0, n1, n2, n3, n4)


def _knn_sc_call(qx, qy, kx, ky, n_pad, m_pad):
    qpw = n_pad // _NW
    ng = qpw // _L
    nkc = m_pad // _L
    bins_sz = m_pad + _L
    mesh = plsc.VectorSubcoreMesh(core_axis_name="c", subcore_axis_name="s")

    @functools.partial(
        pl.kernel,
        out_type=jax.ShapeDtypeStruct((_KNN_K * n_pad,), jnp.float32),
        mesh=mesh,
        compiler_params=pltpu.CompilerParams(needs_layout_passes=False),
        scratch_types=[
            pltpu.VMEM((m_pad,), jnp.float32),
            pltpu.VMEM((m_pad,), jnp.float32),
            pltpu.VMEM((qpw,), jnp.float32),
            pltpu.VMEM((qpw,), jnp.float32),
            pltpu.VMEM((_KNN_K * qpw,), jnp.float32),
            pltpu.VMEM((bins_sz,), jnp.float32),
            pltpu.VMEM((bins_sz,), jnp.float32),
            pltpu.VMEM((_CNTSZ,), jnp.int32),
            pltpu.VMEM((_CNTSZ,), jnp.int32),
            pltpu.VMEM((_CNTSZ,), jnp.int32),
        ],
    )
    def knn_kernel(qx_hbm, qy_hbm, kx_hbm, ky_hbm, out_hbm,
                   kx_v, ky_v, qx_v, qy_v, res_v, bx_v, by_v, cnt_v,
                   st_v, cur_v):
        wid = lax.axis_index("s") * _NC + lax.axis_index("c")
        pltpu.sync_copy(kx_hbm, kx_v)
        pltpu.sync_copy(ky_hbm, ky_v)
        base = wid * qpw
        pltpu.sync_copy(qx_hbm.at[pl.ds(base, qpw)], qx_v)
        pltpu.sync_copy(qy_hbm.at[pl.ds(base, qpw)], qy_v)

        zz = jnp.zeros((_L,), jnp.int32)
        for i in range(_CNTSZ // _L):
            cnt_v[pl.ds(i * _L, _L)] = zz

        gf = jnp.float32(_G)

        def key_cells(c):
            kxc = kx_v[pl.ds(c * _L, _L)]
            kyc = ky_v[pl.ds(c * _L, _L)]
            cxi = jnp.minimum(kxc * gf, 300.0).astype(jnp.int32)
            cyi = jnp.minimum(kyc * gf, 300.0).astype(jnp.int32)
            real = (cxi >= 0) & (cxi < _G) & (cyi >= 0) & (cyi < _G)
            cid = jnp.where(real, cyi * _G + cxi, _NCELL)
            return kxc, kyc, cid, real

        def cnt_body(c, carry):
            _, _, cid, real = key_cells(c)
            dup, last = plsc.scan_count(cid, mask=real)
            cnt = plsc.load_gather(cnt_v, [cid])
            plsc.store_scatter(cnt_v, [cid], cnt + dup, mask=last)
            return carry

        lax.fori_loop(0, nkc, cnt_body, 0)

        run = zz
        for i in range(_NCELL // _L):
            v = cnt_v[pl.ds(i * _L, _L)]
            cs = plsc.cumsum(v)
            st = cs - v + run
            st_v[pl.ds(i * _L, _L)] = st
            cur_v[pl.ds(i * _L, _L)] = st
            run = run + cs[_L - 1]
        st_v[pl.ds(_NCELL, _L)] = run

        def fill_body(c, carry):
            kxc, kyc, cid, real = key_cells(c)
            dup, last = plsc.scan_count(cid, mask=real)
            cur = plsc.load_gather(cur_v, [cid])
            pos = cur + (dup - 1)
            plsc.store_scatter(bx_v, [pos], kxc, mask=real)
            plsc.store_scatter(by_v, [pos], kyc, mask=real)
            plsc.store_scatter(cur_v, [cid], cur + dup, mask=last)
            return carry

        lax.fori_loop(0, nkc, fill_body, 0)

        def scan_range(ts, rbase, ln, nit, qxg, qyg):
            def sbody(it, ts):
                s0 = it * _SCAN_U
                for u in range(_SCAN_U):
                    s = s0 + u
                    m = ln > s
                    idx = jnp.minimum(rbase + s, bins_sz - 1)
                    bxv = plsc.load_gather(bx_v, [idx], mask=m)
                    byv = plsc.load_gather(by_v, [idx], mask=m)
                    dx = qxg - bxv
                    dy = qyg - byv
                    d2 = dx * dx + dy * dy
                    kf = jnp.where(d2 < _PEN_T2, _BIG, d2)
                    kf = jnp.where(m, kf, _SENT)
                    ts = _insert5(ts, kf)
                return ts

            return lax.fori_loop(0, nit, sbody, ts)

        def brute_group(qxg, qyg):
            def chunk_body(c, ts):
                kxc = kx_v[pl.ds(c * _L, _L)]
                kyc = ky_v[pl.ds(c * _L, _L)]
                for i in range(_L):
                    dx = qxg - kxc[i]
                    dy = qyg - kyc[i]
                    d2 = dx * dx + dy * dy
                    kf = jnp.where(d2 < _PEN_T2, _BIG, d2)
                    ts = _insert5(ts, kf)
                return ts

            init = tuple(jnp.full((_L,), _SENT, jnp.float32)
                         for _ in range(_KNN_K))
            return lax.fori_loop(0, nkc, chunk_body, init)

        def group_body(g, carry):
            qxg = qx_v[pl.ds(g * _L, _L)]
            qyg = qy_v[pl.ds(g * _L, _L)]
            cxi = jnp.clip((qxg * gf).astype(jnp.int32), 0, _G - 1)
            cyi = jnp.clip((qyg * gf).astype(jnp.int32), 0, _G - 1)
            ts = tuple(jnp.full((_L,), _SENT, jnp.float32)
                       for _ in range(_KNN_K))
            def row_info(rr, c0, c1, cvalid=None):
                rvalid = (rr >= 0) & (rr < _G)
                if cvalid is not None:
                    rvalid = rvalid & cvalid
                rrc = jnp.where(rvalid, rr, 0) * _G
                lo = plsc.load_gather(st_v, [rrc + c0])
                hi = plsc.load_gather(st_v, [rrc + c1 + 1])
                ln = jnp.where(rvalid, hi - lo, 0)
                nit = (jnp.max(ln) + _SCAN_U - 1) // _SCAN_U
                return lo, ln, nit

            def scan_rows(ts, infos):
                for lo, ln, nit in infos:
                    ts = scan_range(ts, lo, ln, nit, qxg, qyg)
                return ts

            c0 = jnp.maximum(cxi - 1, 0)
            c1 = jnp.minimum(cxi + 1, _G - 1)
            ts = scan_rows(ts, [row_info(cyi + dr, c0, c1)
                                for dr in (-1, 0, 1)])

            fail1 = jnp.any(ts[_KNN_K - 1] > _CELLW2)

            def rescue(t):
                c0r = jnp.maximum(cxi - 2, 0)
                c1r = jnp.minimum(cxi + 2, _G - 1)
                infos = [row_info(cyi + dr, c0r, c1r) for dr in (-2, 2)]
                for dr in (-1, 0, 1):
                    for cc in (cxi - 2, cxi + 2):
                        cv = (cc >= 0) & (cc < _G)
                        ccc = jnp.clip(cc, 0, _G - 1)
                        infos.append(row_info(cyi + dr, ccc, ccc, cvalid=cv))
                return scan_rows(t, infos)

            ts = lax.cond(fail1, rescue, lambda t: t, ts)
            fail2 = jnp.any(ts[_KNN_K - 1] > _CELLW2_2)
            ts = lax.cond(fail2, lambda t: brute_group(qxg, qyg),
                          lambda t: t, ts)
            for i in range(_KNN_K):
                res_v[pl.ds(i * qpw + g * _L, _L)] = ts[i]
            return carry

        lax.fori_loop(0, ng, group_body, 0)
        for i in range(_KNN_K):
            pltpu.sync_copy(res_v.at[pl.ds(i * qpw, qpw)],
                            out_hbm.at[pl.ds(i * n_pad + base, qpw)])

    return knn_kernel(qx, qy, kx, ky)


def _make_loss_tc_kernels(n, rows):
    lanes = 128

    def valid_mask():
        ridx = lax.broadcasted_iota(jnp.int32, (rows, lanes), 0)
        cidx = lax.broadcasted_iota(jnp.int32, (rows, lanes), 1)
        return (ridx * lanes + cidx) < n

    def row(x_ref, i):
        return x_ref[i * rows:(i + 1) * rows, :]

    def loss_pre_kernel(x_ref, reg_ref, sig_ref):
        valid = valid_mask()
        nf = jnp.float32(n)
        st = row(x_ref, 8)
        beta = jnp.float32(0.5)
        reg_sum = jnp.float32(0.0)
        sig_sum = jnp.float32(0.0)
        for pd, ls, g, p in ((row(x_ref, 0), row(x_ref, 2), row(x_ref, 4),
                              row(x_ref, 6)),
                             (row(x_ref, 1), row(x_ref, 3), row(x_ref, 5),
                              row(x_ref, 7))):
            lsc = jnp.clip(ls, _LOG_SIGMA_MIN, _LOG_SIGMA_MAX)
            sq = jnp.exp(lsc)
            gd = (g - p) / st
            diff = pd - gd
            ad = jnp.abs(diff)
            sl1 = jnp.where(ad < beta, 0.5 * diff * diff / beta,
                            ad - 0.5 * beta)
            reg_sum = reg_sum + jnp.sum(jnp.where(valid, sl1, 0.0))
            sigt = lsc + (diff * diff) / (2.0 * sq * sq)
            sig_sum = sig_sum + jnp.sum(jnp.where(valid, sigt, 0.0))
        reg_ref[0, 0] = reg_sum / nf
        sig_ref[0, 0] = sig_sum / nf

    def loss_kl_kernel(x_ref, t_ref, kl_ref):
        valid = valid_mask()
        nf = jnp.float32(n)
        st = row(x_ref, 8)
        dsum = jnp.zeros((rows, lanes), jnp.float32)
        for i in range(_KNN_K):
            s = t_ref[i * rows:(i + 1) * rows, :]
            d = jnp.sqrt(jnp.maximum(s, 1e-12))
            dsum = dsum + jnp.where(s >= 1e11, jnp.float32(1e8), d)
        d_i = dsum / jnp.float32(_KNN_K)
        d_norm = jnp.clip(d_i / st, _PRIOR_DELTA_MIN, _PRIOR_DELTA_MAX)
        sigma_c = jnp.maximum(d_norm, 0.5)
        sigma_p = jnp.maximum(sigma_c, 0.0001)
        kl_sum = jnp.float32(0.0)
        for pd, ls in ((row(x_ref, 0), row(x_ref, 2)),
                       (row(x_ref, 1), row(x_ref, 3))):
            lsc = jnp.clip(ls, _LOG_SIGMA_MIN, _LOG_SIGMA_MAX)
            sq = jnp.exp(lsc)
            kl = (jnp.log(sigma_p / sq)
                  + (sq * sq + pd * pd) / (2.0 * sigma_p * sigma_p) - 0.5)
            kl_sum = kl_sum + jnp.sum(jnp.where(valid, kl, 0.0))
        kl_ref[0, 0] = kl_sum / nf

    return loss_pre_kernel, loss_kl_kernel


def _pad_col(v, n_pad):
    n = v.shape[0]
    if n_pad != n:
        v = jnp.pad(v, (0, n_pad - n))
    return v


def kernel(pred_delta, pred_log_sigma, pos_points, pos_strides, gt_centers,
           gt_centers_list, cur_iter):
    n = pred_delta.shape[0]
    all_gt = gt_centers_list.reshape(-1, 2)
    m = all_gt.shape[0]

    n_pad = ((n + _NW * _L - 1) // (_NW * _L)) * (_NW * _L)
    n_pad = ((n_pad + 1023) // 1024) * 1024
    qx = jnp.pad(gt_centers[:, 0], (0, n_pad - n), constant_values=0.5)
    qy = jnp.pad(gt_centers[:, 1], (0, n_pad - n), constant_values=0.5)
    m_pad = ((m + _L - 1) // _L) * _L
    kx = jnp.pad(all_gt[:, 0], (0, m_pad - m), constant_values=1e9)
    ky = jnp.pad(all_gt[:, 1], (0, m_pad - m), constant_values=1e9)
    top5 = _knn_sc_call(qx, qy, kx, ky, n_pad, m_pad)

    rows = n_pad // 128
    xstk = jnp.stack([
        _pad_col(pred_delta[:, 0], n_pad),
        _pad_col(pred_delta[:, 1], n_pad),
        _pad_col(pred_log_sigma[:, 0], n_pad),
        _pad_col(pred_log_sigma[:, 1], n_pad),
        _pad_col(gt_centers[:, 0], n_pad),
        _pad_col(gt_centers[:, 1], n_pad),
        _pad_col(pos_points[:, 0], n_pad),
        _pad_col(pos_points[:, 1], n_pad),
        _pad_col(pos_strides.astype(jnp.float32), n_pad),
    ]).reshape(9 * rows, 128)
    t5 = top5.reshape(_KNN_K * rows, 128)
    scalar_spec = pl.BlockSpec(memory_space=pltpu.SMEM)
    pre_k, kl_k = _make_loss_tc_kernels(n, rows)
    reg, sig = pl.pallas_call(
        pre_k,
        out_shape=[jax.ShapeDtypeStruct((1, 1), jnp.float32)] * 2,
        in_specs=[pl.BlockSpec(memory_space=pltpu.VMEM)],
        out_specs=[scalar_spec] * 2,
    )(xstk)
    (kl,) = pl.pallas_call(
        kl_k,
        out_shape=[jax.ShapeDtypeStruct((1, 1), jnp.float32)],
        in_specs=[pl.BlockSpec(memory_space=pltpu.VMEM)] * 2,
        out_specs=[scalar_spec],
    )(xstk, t5)

    cur = jnp.asarray(cur_iter, dtype=jnp.float32)
    ratio = jnp.minimum(1.0, (cur - _WARMUP_ITERS) / max(_ANNEAL_ITERS, 1))
    val = _LAMBDA_KL_WARMUP + ratio * (_LAMBDA_KL - _LAMBDA_KL_WARMUP)
    eff_lambda = jnp.where(cur < _WARMUP_ITERS, _LAMBDA_KL_WARMUP,
                           val).astype(jnp.float32)

    return (_LAMBDA_REG * reg[0, 0], _LAMBDA_SIGMA * sig[0, 0],
            eff_lambda * kl[0, 0])


# speedup vs baseline: 1.1511x; 1.0006x over previous
import functools
import jax
import jax.numpy as jnp
import numpy as np
from jax import lax
from jax.experimental import pallas as pl
from jax.experimental.pallas import tpu as pltpu
from jax.experimental.pallas import tpu_sc as plsc

_LAMBDA_REG = 10.0
_LAMBDA_SIGMA = 1.0
_LAMBDA_KL = 0.05
_LAMBDA_KL_WARMUP = 0.005
_KNN_K = 5
_WARMUP_ITERS = 1000
_ANNEAL_ITERS = 3000
_PRIOR_DELTA_MIN = 0.5
_PRIOR_DELTA_MAX = 20.0
_LOG_SIGMA_MIN = -6.0
_LOG_SIGMA_MAX = 4.0

_BIG = np.float32(1e12)
_PEN_T2 = np.float32(1e-4)

_NC = 2
_NS = 16
_NW = _NC * _NS
_L = 16


_G = 32
_NCELL = _G * _G
_CELLW2 = np.float32(1.0 / (_G * _G))
_CELLW2_2 = np.float32(4.0 / (_G * _G))
_SENT = np.float32(3.0e38)
_CNTSZ = ((_NCELL + 1 + _L - 1) // _L) * _L
_SCAN_U = 4


def _insert5(ts, kf):
    t0, t1, t2, t3, t4 = ts
    n4 = jnp.minimum(jnp.maximum(kf, t3), t4)
    n3 = jnp.minimum(jnp.maximum(kf, t2), t3)
    n2 = jnp.minimum(jnp.maximum(kf, t1), t2)
    n1 = jnp.minimum(jnp.maximum(kf, t0), t1)
    n0 = jnp.minimum(kf, t0)
    return (n0, n1, n2, n3, n4)


def _knn_sc_call(qx, qy, kx, ky, n_pad, m_pad):
    qpw = n_pad // _NW
    ng = qpw // _L
    nkc = m_pad // _L
    bins_sz = m_pad + _L
    mesh = plsc.VectorSubcoreMesh(core_axis_name="c", subcore_axis_name="s")

    @functools.partial(
        pl.kernel,
        out_type=jax.ShapeDtypeStruct((_KNN_K * n_pad,), jnp.float32),
        mesh=mesh,
        compiler_params=pltpu.CompilerParams(needs_layout_passes=False,
                                             skip_device_barrier=True),
        scratch_types=[
            pltpu.VMEM((m_pad,), jnp.float32),
            pltpu.VMEM((m_pad,), jnp.float32),
            pltpu.VMEM((qpw,), jnp.float32),
            pltpu.VMEM((qpw,), jnp.float32),
            pltpu.VMEM((_KNN_K * qpw,), jnp.float32),
            pltpu.VMEM((bins_sz,), jnp.float32),
            pltpu.VMEM((bins_sz,), jnp.float32),
            pltpu.VMEM((_CNTSZ,), jnp.int32),
            pltpu.VMEM((_CNTSZ,), jnp.int32),
            pltpu.VMEM((_CNTSZ,), jnp.int32),
        ],
    )
    def knn_kernel(qx_hbm, qy_hbm, kx_hbm, ky_hbm, out_hbm,
                   kx_v, ky_v, qx_v, qy_v, res_v, bx_v, by_v, cnt_v,
                   st_v, cur_v):
        wid = lax.axis_index("s") * _NC + lax.axis_index("c")
        pltpu.sync_copy(kx_hbm, kx_v)
        pltpu.sync_copy(ky_hbm, ky_v)
        base = wid * qpw
        pltpu.sync_copy(qx_hbm.at[pl.ds(base, qpw)], qx_v)
        pltpu.sync_copy(qy_hbm.at[pl.ds(base, qpw)], qy_v)

        zz = jnp.zeros((_L,), jnp.int32)
        for i in range(_CNTSZ // _L):
            cnt_v[pl.ds(i * _L, _L)] = zz

        gf = jnp.float32(_G)

        def key_cells(c):
            kxc = kx_v[pl.ds(c * _L, _L)]
            kyc = ky_v[pl.ds(c * _L, _L)]
            cxi = jnp.minimum(kxc * gf, 300.0).astype(jnp.int32)
            cyi = jnp.minimum(kyc * gf, 300.0).astype(jnp.int32)
            real = (cxi >= 0) & (cxi < _G) & (cyi >= 0) & (cyi < _G)
            cid = jnp.where(real, cyi * _G + cxi, _NCELL)
            return kxc, kyc, cid, real

        def cnt_body(c, carry):
            _, _, cid, real = key_cells(c)
            dup, last = plsc.scan_count(cid, mask=real)
            cnt = plsc.load_gather(cnt_v, [cid])
            plsc.store_scatter(cnt_v, [cid], cnt + dup, mask=last)
            return carry

        lax.fori_loop(0, nkc, cnt_body, 0)

        run = zz
        for i in range(_NCELL // _L):
            v = cnt_v[pl.ds(i * _L, _L)]
            cs = plsc.cumsum(v)
            st = cs - v + run
            st_v[pl.ds(i * _L, _L)] = st
            cur_v[pl.ds(i * _L, _L)] = st
            run = run + cs[_L - 1]
        st_v[pl.ds(_NCELL, _L)] = run

        def fill_body(c, carry):
            kxc, kyc, cid, real = key_cells(c)
            dup, last = plsc.scan_count(cid, mask=real)
            cur = plsc.load_gather(cur_v, [cid])
            pos = cur + (dup - 1)
            plsc.store_scatter(bx_v, [pos], kxc, mask=real)
            plsc.store_scatter(by_v, [pos], kyc, mask=real)
            plsc.store_scatter(cur_v, [cid], cur + dup, mask=last)
            return carry

        lax.fori_loop(0, nkc, fill_body, 0)

        def scan_range(ts, rbase, ln, nit, qxg, qyg):
            def sbody(it, ts):
                s0 = it * _SCAN_U
                for u in range(_SCAN_U):
                    s = s0 + u
                    m = ln > s
                    idx = jnp.minimum(rbase + s, bins_sz - 1)
                    bxv = plsc.load_gather(bx_v, [idx], mask=m)
                    byv = plsc.load_gather(by_v, [idx], mask=m)
                    dx = qxg - bxv
                    dy = qyg - byv
                    d2 = dx * dx + dy * dy
                    kf = jnp.where(d2 < _PEN_T2, _BIG, d2)
                    kf = jnp.where(m, kf, _SENT)
                    ts = _insert5(ts, kf)
                return ts

            return lax.fori_loop(0, nit, sbody, ts)

        def brute_group(qxg, qyg):
            def chunk_body(c, ts):
                kxc = kx_v[pl.ds(c * _L, _L)]
                kyc = ky_v[pl.ds(c * _L, _L)]
                for i in range(_L):
                    dx = qxg - kxc[i]
                    dy = qyg - kyc[i]
                    d2 = dx * dx + dy * dy
                    kf = jnp.where(d2 < _PEN_T2, _BIG, d2)
                    ts = _insert5(ts, kf)
                return ts

            init = tuple(jnp.full((_L,), _SENT, jnp.float32)
                         for _ in range(_KNN_K))
            return lax.fori_loop(0, nkc, chunk_body, init)

        def group_body(g, carry):
            qxg = qx_v[pl.ds(g * _L, _L)]
            qyg = qy_v[pl.ds(g * _L, _L)]
            cxi = jnp.clip((qxg * gf).astype(jnp.int32), 0, _G - 1)
            cyi = jnp.clip((qyg * gf).astype(jnp.int32), 0, _G - 1)
            ts = tuple(jnp.full((_L,), _SENT, jnp.float32)
                       for _ in range(_KNN_K))
            def row_info(rr, c0, c1, cvalid=None):
                rvalid = (rr >= 0) & (rr < _G)
                if cvalid is not None:
                    rvalid = rvalid & cvalid
                rrc = jnp.where(rvalid, rr, 0) * _G
                lo = plsc.load_gather(st_v, [rrc + c0])
                hi = plsc.load_gather(st_v, [rrc + c1 + 1])
                ln = jnp.where(rvalid, hi - lo, 0)
                nit = (jnp.max(ln) + _SCAN_U - 1) // _SCAN_U
                return lo, ln, nit

            def scan_rows(ts, infos):
                for lo, ln, nit in infos:
                    ts = scan_range(ts, lo, ln, nit, qxg, qyg)
                return ts

            c0 = jnp.maximum(cxi - 1, 0)
            c1 = jnp.minimum(cxi + 1, _G - 1)
            ts = scan_rows(ts, [row_info(cyi + dr, c0, c1)
                                for dr in (-1, 0, 1)])

            fail1 = jnp.any(ts[_KNN_K - 1] > _CELLW2)

            def rescue(t):
                c0r = jnp.maximum(cxi - 2, 0)
                c1r = jnp.minimum(cxi + 2, _G - 1)
                infos = [row_info(cyi + dr, c0r, c1r) for dr in (-2, 2)]
                for dr in (-1, 0, 1):
                    for cc in (cxi - 2, cxi + 2):
                        cv = (cc >= 0) & (cc < _G)
                        ccc = jnp.clip(cc, 0, _G - 1)
                        infos.append(row_info(cyi + dr, ccc, ccc, cvalid=cv))
                return scan_rows(t, infos)

            ts = lax.cond(fail1, rescue, lambda t: t, ts)
            fail2 = jnp.any(ts[_KNN_K - 1] > _CELLW2_2)
            ts = lax.cond(fail2, lambda t: brute_group(qxg, qyg),
                          lambda t: t, ts)
            for i in range(_KNN_K):
                res_v[pl.ds(i * qpw + g * _L, _L)] = ts[i]
            return carry

        lax.fori_loop(0, ng, group_body, 0)
        for i in range(_KNN_K):
            pltpu.sync_copy(res_v.at[pl.ds(i * qpw, qpw)],
                            out_hbm.at[pl.ds(i * n_pad + base, qpw)])

    return knn_kernel(qx, qy, kx, ky)


def _make_loss_tc_kernels(n, rows):
    lanes = 128

    def valid_mask():
        ridx = lax.broadcasted_iota(jnp.int32, (rows, lanes), 0)
        cidx = lax.broadcasted_iota(jnp.int32, (rows, lanes), 1)
        return (ridx * lanes + cidx) < n

    def row(x_ref, i):
        return x_ref[i * rows:(i + 1) * rows, :]

    def loss_pre_kernel(x_ref, reg_ref, sig_ref):
        valid = valid_mask()
        nf = jnp.float32(n)
        st = row(x_ref, 8)
        beta = jnp.float32(0.5)
        reg_sum = jnp.float32(0.0)
        sig_sum = jnp.float32(0.0)
        for pd, ls, g, p in ((row(x_ref, 0), row(x_ref, 2), row(x_ref, 4),
                              row(x_ref, 6)),
                             (row(x_ref, 1), row(x_ref, 3), row(x_ref, 5),
                              row(x_ref, 7))):
            lsc = jnp.clip(ls, _LOG_SIGMA_MIN, _LOG_SIGMA_MAX)
            sq = jnp.exp(lsc)
            gd = (g - p) / st
            diff = pd - gd
            ad = jnp.abs(diff)
            sl1 = jnp.where(ad < beta, 0.5 * diff * diff / beta,
                            ad - 0.5 * beta)
            reg_sum = reg_sum + jnp.sum(jnp.where(valid, sl1, 0.0))
            sigt = lsc + (diff * diff) / (2.0 * sq * sq)
            sig_sum = sig_sum + jnp.sum(jnp.where(valid, sigt, 0.0))
        reg_ref[0, 0] = reg_sum / nf
        sig_ref[0, 0] = sig_sum / nf

    def loss_kl_kernel(x_ref, t_ref, kl_ref):
        valid = valid_mask()
        nf = jnp.float32(n)
        st = row(x_ref, 8)
        dsum = jnp.zeros((rows, lanes), jnp.float32)
        for i in range(_KNN_K):
            s = t_ref[i * rows:(i + 1) * rows, :]
            d = jnp.sqrt(jnp.maximum(s, 1e-12))
            dsum = dsum + jnp.where(s >= 1e11, jnp.float32(1e8), d)
        d_i = dsum / jnp.float32(_KNN_K)
        d_norm = jnp.clip(d_i / st, _PRIOR_DELTA_MIN, _PRIOR_DELTA_MAX)
        sigma_c = jnp.maximum(d_norm, 0.5)
        sigma_p = jnp.maximum(sigma_c, 0.0001)
        kl_sum = jnp.float32(0.0)
        for pd, ls in ((row(x_ref, 0), row(x_ref, 2)),
                       (row(x_ref, 1), row(x_ref, 3))):
            lsc = jnp.clip(ls, _LOG_SIGMA_MIN, _LOG_SIGMA_MAX)
            sq = jnp.exp(lsc)
            kl = (jnp.log(sigma_p / sq)
                  + (sq * sq + pd * pd) / (2.0 * sigma_p * sigma_p) - 0.5)
            kl_sum = kl_sum + jnp.sum(jnp.where(valid, kl, 0.0))
        kl_ref[0, 0] = kl_sum / nf

    return loss_pre_kernel, loss_kl_kernel


def _pad_col(v, n_pad):
    n = v.shape[0]
    if n_pad != n:
        v = jnp.pad(v, (0, n_pad - n))
    return v


def kernel(pred_delta, pred_log_sigma, pos_points, pos_strides, gt_centers,
           gt_centers_list, cur_iter):
    n = pred_delta.shape[0]
    all_gt = gt_centers_list.reshape(-1, 2)
    m = all_gt.shape[0]

    n_pad = ((n + _NW * _L - 1) // (_NW * _L)) * (_NW * _L)
    n_pad = ((n_pad + 1023) // 1024) * 1024
    qx = jnp.pad(gt_centers[:, 0], (0, n_pad - n), constant_values=0.5)
    qy = jnp.pad(gt_centers[:, 1], (0, n_pad - n), constant_values=0.5)
    m_pad = ((m + _L - 1) // _L) * _L
    kx = jnp.pad(all_gt[:, 0], (0, m_pad - m), constant_values=1e9)
    ky = jnp.pad(all_gt[:, 1], (0, m_pad - m), constant_values=1e9)
    top5 = _knn_sc_call(qx, qy, kx, ky, n_pad, m_pad)

    rows = n_pad // 128
    xstk = jnp.stack([
        _pad_col(pred_delta[:, 0], n_pad),
        _pad_col(pred_delta[:, 1], n_pad),
        _pad_col(pred_log_sigma[:, 0], n_pad),
        _pad_col(pred_log_sigma[:, 1], n_pad),
        _pad_col(gt_centers[:, 0], n_pad),
        _pad_col(gt_centers[:, 1], n_pad),
        _pad_col(pos_points[:, 0], n_pad),
        _pad_col(pos_points[:, 1], n_pad),
        _pad_col(pos_strides.astype(jnp.float32), n_pad),
    ]).reshape(9 * rows, 128)
    t5 = top5.reshape(_KNN_K * rows, 128)
    scalar_spec = pl.BlockSpec(memory_space=pltpu.SMEM)
    pre_k, kl_k = _make_loss_tc_kernels(n, rows)
    reg, sig = pl.pallas_call(
        pre_k,
        out_shape=[jax.ShapeDtypeStruct((1, 1), jnp.float32)] * 2,
        in_specs=[pl.BlockSpec(memory_space=pltpu.VMEM)],
        out_specs=[scalar_spec] * 2,
    )(xstk)
    (kl,) = pl.pallas_call(
        kl_k,
        out_shape=[jax.ShapeDtypeStruct((1, 1), jnp.float32)],
        in_specs=[pl.BlockSpec(memory_space=pltpu.VMEM)] * 2,
        out_specs=[scalar_spec],
    )(xstk, t5)

    cur = jnp.asarray(cur_iter, dtype=jnp.float32)
    ratio = jnp.minimum(1.0, (cur - _WARMUP_ITERS) / max(_ANNEAL_ITERS, 1))
    val = _LAMBDA_KL_WARMUP + ratio * (_LAMBDA_KL - _LAMBDA_KL_WARMUP)
    eff_lambda = jnp.where(cur < _WARMUP_ITERS, _LAMBDA_KL_WARMUP,
                           val).astype(jnp.float32)

    return (_LAMBDA_REG * reg[0, 0], _LAMBDA_SIGMA * sig[0, 0],
            eff_lambda * kl[0, 0])
